# baseline (speedup 1.0000x reference)
import functools

import jax
import jax.numpy as jnp
from jax import lax
from jax.experimental import pallas as pl
from jax.experimental.pallas import tpu as pltpu
from jax.experimental.pallas import tpu_sc as plsc

_L = 16
_MIN32 = -2147483648


def _scan_hist2(hist, krem0, krem1, spread=False):
    iota = lax.iota(jnp.int32, _L)
    state = []
    for krem in (krem0, krem1):
        state.append([jnp.int32(0), jnp.int32(256), jnp.int32(0), jnp.int32(0), krem])
    for vi in range(256 // _L):
        for c in (0, 1):
            carry, dstar, before, hsel, krem = state[c]
            if spread:
                base = c * 1024 + vi * _L
                hv = (hist[pl.ds(base, _L)] + hist[pl.ds(base + 256, _L)]
                      + hist[pl.ds(base + 512, _L)] + hist[pl.ds(base + 768, _L)])
            else:
                hv = hist[pl.ds(2048 + c * 256 + vi * _L, _L)]
            g = carry + plsc.cumsum(hv)
            nb = jnp.sum((g < krem).astype(jnp.int32), axis=0)
            sel = iota == nb
            gd = jnp.sum(jnp.where(sel, g, 0), axis=0)
            hd = jnp.sum(jnp.where(sel, hv, 0), axis=0)
            first = jnp.logical_and(nb < _L, dstar == 256)
            state[c] = [
                carry + jnp.sum(hv, axis=0),
                jnp.where(first, vi * _L + nb, dstar),
                jnp.where(first, gd - hd, before),
                jnp.where(first, hd, hsel),
                krem,
            ]
    return [(s[1], s[2], s[3]) for s in state]


def _pair(n, xv, keyv, outv, hist):
    k = n // 2
    nv = n // _L
    ones = jnp.ones((_L,), jnp.int32)
    zeros = jnp.zeros((_L,), jnp.int32)

    p = [jnp.int32(0), jnp.int32(0)]
    krem = [jnp.int32(k), jnp.int32(k)]
    esel = [jnp.int32(0), jnp.int32(0)]

    lane4 = lax.mul(lax.iota(jnp.int32, _L) & 3, jnp.int32(256))
    for li, s in enumerate((24, 16, 8, 0)):
        nzero = 2048 if li == 0 else 512
        for i in range(nzero // _L):
            hist[pl.ds(i * _L, _L)] = zeros

        if li == 0:
            @plsc.parallel_loop(0, nv, unroll=8)
            def _(i):
                for c in (0, 1):
                    xvec = xv[c, pl.ds(i * _L, _L)]
                    ib = lax.bitcast_convert_type(xvec, jnp.int32)
                    asc = jnp.where(ib >= 0, ib, -(ib & jnp.int32(0x7FFFFFFF)))
                    kb = (~asc) ^ _MIN32
                    keyv[c, pl.ds(i * _L, _L)] = kb
                    d = (lax.shift_right_logical(kb, 24) & 255) + (c * 1024) + lane4
                    plsc.addupdate_scatter(hist, [d], ones)
        else:
            @plsc.parallel_loop(0, nv, unroll=8)
            def _(i, _s=s, _p=tuple(p)):
                for c in (0, 1):
                    kb = keyv[c, pl.ds(i * _L, _L)]
                    pm = lax.shift_right_logical(kb, _s + 8) == _p[c]
                    d = (lax.shift_right_logical(kb, _s) & 255) + 2048 + c * 256
                    plsc.addupdate_scatter(hist, [d], ones, mask=pm)

        res = _scan_hist2(hist, krem[0], krem[1], spread=(li == 0))
        for c in (0, 1):
            dstar, nbefore, hsel = res[c]
            krem[c] = krem[c] - nbefore
            p[c] = lax.shift_left(p[c], 8) | dstar
            esel[c] = hsel

    ts = [p[0] ^ _MIN32, p[1] ^ _MIN32]
    m = krem
    both_fast = jnp.logical_and(m[0] == esel[0], m[1] == esel[1])

    @pl.when(both_fast)
    def _():
        @plsc.parallel_loop(0, nv, unroll=8)
        def _(i):
            for c in (0, 1):
                ks = keyv[c, pl.ds(i * _L, _L)] ^ _MIN32
                outv[c, pl.ds(i * _L, _L)] = jnp.where(
                    ks <= ts[c], jnp.float32(1.0), jnp.float32(-1.0))

    @pl.when(jnp.logical_not(both_fast))
    def _():
        for c in (0, 1):
            def body(i, cnt, _c=c):
                ks = keyv[_c, pl.ds(i * _L, _L)] ^ _MIN32
                eqm = ks == ts[_c]
                eqi = eqm.astype(jnp.int32)
                pos = cnt + plsc.cumsum(eqi)
                take = (ks < ts[_c]) | (eqm & (pos <= m[_c]))
                outv[_c, pl.ds(i * _L, _L)] = jnp.where(
                    take, jnp.float32(1.0), jnp.float32(-1.0))
                return cnt + jnp.sum(eqi, axis=0)
            lax.fori_loop(0, nv, body, jnp.int32(0))


def _make_sc(n, d):
    cols = d // 32
    mesh = plsc.VectorSubcoreMesh(core_axis_name="c", subcore_axis_name="s")

    @functools.partial(
        pl.kernel,
        mesh=mesh,
        out_type=jax.ShapeDtypeStruct((d, n), jnp.float32),
        compiler_params=pltpu.CompilerParams(needs_layout_passes=False),
        scratch_types=[
            pltpu.VMEM((cols, n), jnp.float32),
            pltpu.VMEM((cols, n), jnp.int32),
            pltpu.VMEM((cols, n), jnp.float32),
            pltpu.VMEM((2560,), jnp.int32),
        ],
    )
    def run(x_hbm, out_hbm, xv, keyv, outv, hist):
        wid = lax.axis_index("s") * 2 + lax.axis_index("c")
        base = wid * cols
        pltpu.sync_copy(x_hbm.at[pl.ds(base, cols)], xv)
        _pair(n, xv, keyv, outv, hist)
        pltpu.sync_copy(outv, out_hbm.at[pl.ds(base, cols)])

    return run


@jax.jit
def kernel(x):
    n, d = x.shape
    out_t = _make_sc(n, d)(x.T)
    return out_t.T


# speedup vs baseline: 1.1195x; 1.1195x over previous
import functools

import jax
import jax.numpy as jnp
from jax import lax
from jax.experimental import pallas as pl
from jax.experimental.pallas import tpu as pltpu
from jax.experimental.pallas import tpu_sc as plsc

_L = 16
_CH = 8
_MIN32 = -2147483648


def _scan_hist2(hist, krem0, krem1):
    iota = lax.iota(jnp.int32, _L)
    state = []
    for krem in (krem0, krem1):
        state.append([jnp.int32(0), jnp.int32(256), jnp.int32(0), jnp.int32(0), krem])
    for vi in range(256 // _L):
        for c in (0, 1):
            carry, dstar, before, hsel, krem = state[c]
            hv = hist[pl.ds(c * 256 + vi * _L, _L)]
            g = carry + plsc.cumsum(hv)
            nb = jnp.sum((g < krem).astype(jnp.int32), axis=0)
            sel = iota == nb
            gd = jnp.sum(jnp.where(sel, g, 0), axis=0)
            hd = jnp.sum(jnp.where(sel, hv, 0), axis=0)
            first = jnp.logical_and(nb < _L, dstar == 256)
            state[c] = [
                carry + jnp.sum(hv, axis=0),
                jnp.where(first, vi * _L + nb, dstar),
                jnp.where(first, gd - hd, before),
                jnp.where(first, hd, hsel),
                krem,
            ]
    return [(s[1], s[2], s[3]) for s in state]


def _make_sc(n, d):
    cols = d // 32
    k = n // 2
    nv = n // _L
    chunk = n // _CH
    cnv = chunk // _L
    mesh = plsc.VectorSubcoreMesh(core_axis_name="c", subcore_axis_name="s")

    @functools.partial(
        pl.kernel,
        mesh=mesh,
        out_type=jax.ShapeDtypeStruct((d, n), jnp.float32),
        compiler_params=pltpu.CompilerParams(needs_layout_passes=False),
        scratch_types=[
            pltpu.VMEM((cols, n), jnp.float32),
            pltpu.VMEM((cols, n), jnp.int32),
            pltpu.VMEM((cols, n), jnp.float32),
            pltpu.VMEM((512,), jnp.int32),
            pltpu.SemaphoreType.DMA,
            pltpu.SemaphoreType.DMA,
            pltpu.SemaphoreType.DMA,
        ],
    )
    def run(x_hbm, out_hbm, xv, keyv, outv, hist, sem_a, sem_b, sem_o):
        wid = lax.axis_index("s") * 2 + lax.axis_index("c")
        base = wid * cols
        sems = (sem_a, sem_b)

        def in_copy(g):
            return pltpu.make_async_copy(
                x_hbm.at[pl.ds(base, cols), pl.ds(g * chunk, chunk)],
                xv.at[:, pl.ds(g * chunk, chunk)],
                sems[g % 2],
            )

        def out_copy(g):
            return pltpu.make_async_copy(
                outv.at[:, pl.ds(g * chunk, chunk)],
                out_hbm.at[pl.ds(base, cols), pl.ds(g * chunk, chunk)],
                sem_o,
            )

        ones = jnp.ones((_L,), jnp.int32)
        zeros = jnp.zeros((_L,), jnp.int32)

        p = [jnp.int32(0), jnp.int32(0)]
        krem = [jnp.int32(k), jnp.int32(k)]
        esel = [jnp.int32(0), jnp.int32(0)]

        for li, s in enumerate((24, 16, 8, 0)):
            for i in range(512 // _L):
                hist[pl.ds(i * _L, _L)] = zeros

            if li == 0:
                in_copy(0).start()
                in_copy(1).start()
                for g in range(_CH):
                    in_copy(g).wait()
                    if g + 2 < _CH:
                        in_copy(g + 2).start()

                    @plsc.parallel_loop(g * cnv, (g + 1) * cnv, unroll=8)
                    def _(i):
                        for c in (0, 1):
                            xvec = xv[c, pl.ds(i * _L, _L)]
                            ib = lax.bitcast_convert_type(xvec, jnp.int32)
                            asc = jnp.where(ib >= 0, ib,
                                            -(ib & jnp.int32(0x7FFFFFFF)))
                            kb = (~asc) ^ _MIN32
                            keyv[c, pl.ds(i * _L, _L)] = kb
                            d_ = (lax.shift_right_logical(kb, 24) & 255) + c * 256
                            plsc.addupdate_scatter(hist, [d_], ones)
            else:
                @plsc.parallel_loop(0, nv, unroll=8)
                def _(i, _s=s, _p=tuple(p)):
                    for c in (0, 1):
                        kb = keyv[c, pl.ds(i * _L, _L)]
                        pm = lax.shift_right_logical(kb, _s + 8) == _p[c]
                        d_ = (lax.shift_right_logical(kb, _s) & 255) + c * 256
                        plsc.addupdate_scatter(hist, [d_], ones, mask=pm)

            res = _scan_hist2(hist, krem[0], krem[1])
            for c in (0, 1):
                dstar, nbefore, hsel = res[c]
                krem[c] = krem[c] - nbefore
                p[c] = lax.shift_left(p[c], 8) | dstar
                esel[c] = hsel

        ts = [p[0] ^ _MIN32, p[1] ^ _MIN32]
        m = krem
        both_fast = jnp.logical_and(m[0] == esel[0], m[1] == esel[1])

        @pl.when(both_fast)
        def _():
            for g in range(_CH):
                @plsc.parallel_loop(g * cnv, (g + 1) * cnv, unroll=8)
                def _(i):
                    for c in (0, 1):
                        ks = keyv[c, pl.ds(i * _L, _L)] ^ _MIN32
                        outv[c, pl.ds(i * _L, _L)] = jnp.where(
                            ks <= ts[c], jnp.float32(1.0), jnp.float32(-1.0))
                out_copy(g).start()
            for g in range(_CH):
                out_copy(g).wait()

        @pl.when(jnp.logical_not(both_fast))
        def _():
            for c in (0, 1):
                def body(i, cnt, _c=c):
                    ks = keyv[_c, pl.ds(i * _L, _L)] ^ _MIN32
                    eqm = ks == ts[_c]
                    eqi = eqm.astype(jnp.int32)
                    pos = cnt + plsc.cumsum(eqi)
                    take = (ks < ts[_c]) | (eqm & (pos <= m[_c]))
                    outv[_c, pl.ds(i * _L, _L)] = jnp.where(
                        take, jnp.float32(1.0), jnp.float32(-1.0))
                    return cnt + jnp.sum(eqi, axis=0)
                lax.fori_loop(0, nv, body, jnp.int32(0))
            pltpu.sync_copy(outv, out_hbm.at[pl.ds(base, cols)])

    return run


@jax.jit
def kernel(x):
    n, d = x.shape
    out_t = _make_sc(n, d)(x.T)
    return out_t.T
